# Initial kernel scaffold; baseline (speedup 1.0000x reference)
import jax
import jax.numpy as jnp
from jax.experimental import pallas as pl


def kernel(word, word_embeddings, grammar_preterminates, W1, b1):
    raise NotImplementedError("write your pallas kernel here")


# speedup vs baseline: 6.0739x; 6.0739x over previous
import functools

import jax
import jax.numpy as jnp
from jax import lax
from jax.experimental import pallas as pl
from jax.experimental.pallas import tpu as pltpu
from jax.experimental.pallas import tpu_sc as plsc

N_TOK = 32768
D_EMB = 512
S_DIM = 256

NC = 2
NS = 16
NW = NC * NS
B_PER_W = N_TOK // NW
CH = 128
N_CHUNK = B_PER_W // CH

_sc_mesh = plsc.VectorSubcoreMesh(core_axis_name="c", subcore_axis_name="s")


@functools.partial(
    pl.kernel,
    out_type=jax.ShapeDtypeStruct((N_TOK, D_EMB), jnp.float32),
    mesh=_sc_mesh,
    scratch_types=[
        pltpu.VMEM((B_PER_W,), jnp.int32),
        pltpu.VMEM((CH, D_EMB), jnp.float32),
        pltpu.SemaphoreType.DMA,
    ],
)
def _sc_gather(word_hbm, table_hbm, out_hbm, idx_v, rows_v, sem):
    wid = lax.axis_index("s") * NC + lax.axis_index("c")
    base = wid * B_PER_W
    pltpu.sync_copy(word_hbm.at[pl.ds(base, B_PER_W)], idx_v)
    for c in range(N_CHUNK):
        pltpu.async_copy(
            table_hbm.at[idx_v.at[pl.ds(c * CH, CH)]], rows_v, sem
        ).wait()
        pltpu.sync_copy(rows_v, out_hbm.at[pl.ds(base + c * CH, CH)])


BM = 2048


def _mlp_body(x_ref, w_ref, b_ref, o_ref):
    o_ref[...] = jnp.tanh(
        jnp.dot(x_ref[...], w_ref[...], preferred_element_type=jnp.float32)
        + b_ref[...]
    )


_tc_mlp = pl.pallas_call(
    _mlp_body,
    grid=(N_TOK // BM,),
    in_specs=[
        pl.BlockSpec((BM, D_EMB), lambda i: (i, 0)),
        pl.BlockSpec((D_EMB, S_DIM), lambda i: (0, 0)),
        pl.BlockSpec((1, S_DIM), lambda i: (0, 0)),
    ],
    out_specs=pl.BlockSpec((BM, S_DIM), lambda i: (i, 0)),
    out_shape=jax.ShapeDtypeStruct((N_TOK, S_DIM), jnp.float32),
)


def kernel(word, word_embeddings, grammar_preterminates, W1, b1):
    del grammar_preterminates
    emb = _sc_gather(word.astype(jnp.int32), word_embeddings)
    return _tc_mlp(emb, W1, b1.reshape(1, S_DIM))


# speedup vs baseline: 6.3882x; 1.0517x over previous
import functools

import jax
import jax.numpy as jnp
from jax import lax
from jax.experimental import pallas as pl
from jax.experimental.pallas import tpu as pltpu
from jax.experimental.pallas import tpu_sc as plsc

N_TOK = 32768
D_EMB = 512
S_DIM = 256

NC = 2
NS = 16
NW = NC * NS
B_PER_W = N_TOK // NW
CH = 64
N_CHUNK = B_PER_W // CH

_sc_mesh = plsc.VectorSubcoreMesh(core_axis_name="c", subcore_axis_name="s")


@functools.partial(
    pl.kernel,
    out_type=jax.ShapeDtypeStruct((N_TOK, D_EMB), jnp.float32),
    mesh=_sc_mesh,
    scratch_types=[
        pltpu.VMEM((B_PER_W,), jnp.int32),
        pltpu.VMEM((CH, D_EMB), jnp.float32),
        pltpu.VMEM((CH, D_EMB), jnp.float32),
        pltpu.SemaphoreType.DMA,
        pltpu.SemaphoreType.DMA,
    ],
)
def _sc_gather(word_hbm, table_hbm, out_hbm, idx_v, rows_a, rows_b, gsem, ssem):
    wid = lax.axis_index("s") * NC + lax.axis_index("c")
    base = wid * B_PER_W
    pltpu.sync_copy(word_hbm.at[pl.ds(base, B_PER_W)], idx_v)
    bufs = (rows_a, rows_b)
    gathers = [None] * N_CHUNK
    stores = [None] * N_CHUNK
    for c in range(min(2, N_CHUNK)):
        gathers[c] = pltpu.async_copy(
            table_hbm.at[idx_v.at[pl.ds(c * CH, CH)]], bufs[c % 2], gsem
        )
    for c in range(N_CHUNK):
        buf = bufs[c % 2]
        gathers[c].wait()
        stores[c] = pltpu.async_copy(
            buf, out_hbm.at[pl.ds(base + c * CH, CH)], ssem
        )
        if c + 2 < N_CHUNK:
            stores[c].wait()
            gathers[c + 2] = pltpu.async_copy(
                table_hbm.at[idx_v.at[pl.ds((c + 2) * CH, CH)]], buf, gsem
            )
    stores[N_CHUNK - 2].wait()
    stores[N_CHUNK - 1].wait()


BM = 2048


def _mlp_body(x_ref, w_ref, b_ref, o_ref):
    o_ref[...] = jnp.tanh(
        jnp.dot(x_ref[...], w_ref[...], preferred_element_type=jnp.float32)
        + b_ref[...]
    )


_tc_mlp = pl.pallas_call(
    _mlp_body,
    grid=(N_TOK // BM,),
    in_specs=[
        pl.BlockSpec((BM, D_EMB), lambda i: (i, 0)),
        pl.BlockSpec((D_EMB, S_DIM), lambda i: (0, 0)),
        pl.BlockSpec((1, S_DIM), lambda i: (0, 0)),
    ],
    out_specs=pl.BlockSpec((BM, S_DIM), lambda i: (i, 0)),
    out_shape=jax.ShapeDtypeStruct((N_TOK, S_DIM), jnp.float32),
)


def kernel(word, word_embeddings, grammar_preterminates, W1, b1):
    del grammar_preterminates
    emb = _sc_gather(word.astype(jnp.int32), word_embeddings)
    return _tc_mlp(emb, W1, b1.reshape(1, S_DIM))
